# baseline (speedup 1.0000x reference)
import functools

import jax
import jax.numpy as jnp
from jax import lax
from jax.experimental import pallas as pl
from jax.experimental.pallas import tpu as pltpu
from jax.experimental.pallas import tpu_sc as plsc

N_USERS = 20000
N_ITEMS = 30000
NN = N_USERS + N_ITEMS
D = 64
HD = 32
NE = 800000

NS = 16
CH = 128
NB = NE // CH
KTRIP = 99
ZR = 40
NZB = NN // ZR
ZB_PT = -(-NZB // NS)
NG = CH // 16


def _make_spmm(swap):
    mesh = plsc.VectorSubcoreMesh(core_axis_name="c", subcore_axis_name="s")

    @functools.partial(
        pl.kernel,
        mesh=mesh,
        compiler_params=pltpu.CompilerParams(use_tc_tiling_on_sc=False),
        out_type=jax.ShapeDtypeStruct((2 * NN, HD), jnp.float32),
        scratch_types=[
            pltpu.VMEM_SHARED((NN, HD), jnp.float32),
            pltpu.VMEM((CH,), jnp.int32),
            pltpu.VMEM((CH,), jnp.int32),
            pltpu.VMEM((CH,), jnp.int32),
            pltpu.VMEM((CH,), jnp.int32),
            pltpu.VMEM((CH,), jnp.int32),
            pltpu.VMEM((CH,), jnp.int32),
            pltpu.VMEM((CH,), jnp.int32),
            pltpu.VMEM((CH,), jnp.int32),
            pltpu.VMEM((CH,), jnp.float32),
            pltpu.VMEM((CH,), jnp.float32),
            pltpu.VMEM((CH,), jnp.float32),
            pltpu.VMEM((CH,), jnp.float32),
            pltpu.VMEM((CH, HD), jnp.float32),
            pltpu.VMEM((CH, HD), jnp.float32),
            pltpu.VMEM((CH, HD), jnp.float32),
            pltpu.VMEM((CH, HD), jnp.float32),
            pltpu.VMEM((CH, HD), jnp.float32),
            pltpu.VMEM((CH, HD), jnp.float32),
            pltpu.VMEM((ZR, HD), jnp.float32),
            pltpu.SemaphoreType.DMA,
            pltpu.SemaphoreType.DMA,
            pltpu.SemaphoreType.DMA,
            pltpu.SemaphoreType.DMA,
            pltpu.SemaphoreType.DMA,
            pltpu.SemaphoreType.DMA,
            pltpu.SemaphoreType.DMA,
            pltpu.SemaphoreType.DMA,
            pltpu.SemaphoreType.DMA,
            pltpu.SemaphoreType.DMA,
            pltpu.SemaphoreType.DMA,
            pltpu.SemaphoreType.DMA,
        ],
    )
    def spmm(adj_hbm, eval_hbm, x_hbm, out_hbm, acc,
             sb0, sb1, sb2, sb3, db0, db1, db2, db3, vb0, vb1, vb2, vb3,
             rw0, rw1, rw2, rw3, ro0, ro1, zb_v,
             si0, si1, si2, si3, sg0, sg1, sg2, sg3, ss0, ss1, ss2, ss3):
        c = lax.axis_index("c")
        s = lax.axis_index("s")
        gbase = c * NN
        sbs = [sb0, sb1, sb2, sb3]
        dbs = [db0, db1, db2, db3]
        vbs = [vb0, vb1, vb2, vb3]
        rws = [rw0, rw1, rw2, rw3]
        ros = [ro0, ro1]
        sis = [si0, si1, si2, si3]
        sgs = [sg0, sg1, sg2, sg3]
        sss = [ss0, ss1, ss2, ss3]

        def valid(k):
            return s + NS * k < NB

        srow = 1 if swap else 0
        drow = 0 if swap else 1

        def idx_start(k, q):
            @pl.when(valid(k))
            def _():
                e0 = (s + NS * k) * CH
                pltpu.make_async_copy(adj_hbm.at[pl.ds(srow * NE + e0, CH)],
                                      sbs[q], sis[q]).start()
                pltpu.make_async_copy(adj_hbm.at[pl.ds(drow * NE + e0, CH)],
                                      dbs[q], sis[q]).start()
                pltpu.make_async_copy(eval_hbm.at[pl.ds(e0, CH)],
                                      vbs[q], sis[q]).start()

        def idx_wait_adjust(k, q):
            @pl.when(valid(k))
            def _():
                e0 = (s + NS * k) * CH
                pltpu.make_async_copy(adj_hbm.at[pl.ds(srow * NE + e0, CH)],
                                      sbs[q], sis[q]).wait()
                pltpu.make_async_copy(adj_hbm.at[pl.ds(drow * NE + e0, CH)],
                                      dbs[q], sis[q]).wait()
                pltpu.make_async_copy(eval_hbm.at[pl.ds(e0, CH)],
                                      vbs[q], sis[q]).wait()
                for g in range(NG):
                    sl = pl.ds(g * 16, 16)
                    sbs[q][sl] = sbs[q][sl] + gbase

        def gather_start(k, q):
            @pl.when(valid(k))
            def _():
                pltpu.make_async_copy(x_hbm.at[sbs[q]],
                                      rws[q], sgs[q]).start()

        def gather_wait(k, q):
            @pl.when(valid(k))
            def _():
                pltpu.make_async_copy(x_hbm.at[sbs[q]],
                                      rws[q], sgs[q]).wait()

        def scale(k, q):
            @pl.when(valid(k))
            def _():
                rw = rws[q]
                ro = ros[q % 2]
                vb = vbs[q]
                for g in range(NG):
                    v16 = vb[pl.ds(g * 16, 16)]
                    for i in range(16):
                        r = g * 16 + i
                        vi = v16[i]
                        ro[r, pl.ds(0, 16)] = rw[r, pl.ds(0, 16)] * vi
                        ro[r, pl.ds(16, 16)] = rw[r, pl.ds(16, 16)] * vi

        def scatter_start(k, q):
            @pl.when(valid(k))
            def _():
                pltpu.make_async_copy(ros[q % 2], acc.at[dbs[q]],
                                      sss[q]).start(add=True)

        def scatter_wait(k, q, pred):
            @pl.when(pred)
            def _():
                pltpu.make_async_copy(ros[q % 2], acc.at[dbs[q]],
                                      sss[q]).wait()

        idx_start(0, 0)
        idx_start(1, 1)
        idx_start(2, 2)

        def zz(i, carry):
            zb_v[i, pl.ds(0, 16)] = jnp.zeros((16,), jnp.float32)
            zb_v[i, pl.ds(16, 16)] = jnp.zeros((16,), jnp.float32)
            return carry
        lax.fori_loop(0, ZR, zz, 0)

        def zacc(t, carry):
            b = s + t * NS
            @pl.when(b < NZB)
            def _():
                pltpu.sync_copy(zb_v, acc.at[pl.ds(b * ZR, ZR)])
            return carry
        lax.fori_loop(0, ZB_PT, zacc, 0)

        idx_wait_adjust(0, 0)
        gather_start(0, 0)
        idx_wait_adjust(1, 1)
        gather_start(1, 1)
        plsc.subcore_barrier()

        def sub(k, q):
            q1 = (q + 1) % 4
            q2 = (q + 2) % 4
            q3 = (q + 3) % 4
            gather_wait(k, q)
            scale(k, q)
            scatter_start(k, q)
            scatter_wait(k - 1, q3, (k > 0) & valid(k - 1))
            idx_start(k + 3, q3)
            idx_wait_adjust(k + 2, q2)
            gather_start(k + 2, q2)

        def quad(p, carry):
            k = 4 * p
            sub(k, 0)
            sub(k + 1, 1)
            sub(k + 2, 2)
            sub(k + 3, 3)
            return carry
        lax.fori_loop(0, KTRIP, quad, 0)
        plsc.subcore_barrier()

        def wout(t, carry):
            b = s + t * NS
            @pl.when(b < NZB)
            def _():
                pltpu.sync_copy(acc.at[pl.ds(b * ZR, ZR)],
                                out_hbm.at[pl.ds(gbase + b * ZR, ZR)])
            return carry
        lax.fori_loop(0, ZB_PT, wout, 0)

    return spmm


_SPMM_F = _make_spmm(False)
_SPMM_B = _make_spmm(True)

_BR = 1000
_NBLK = NN // _BR
_NU_BLK = N_USERS // _BR
_BR4 = 2000
_NBLK4 = NN // _BR4


def _stage1(ego, d_scaled, Wu, Wi):
    d2 = d_scaled.reshape(NN, 1)

    def body(ego_ref, d_ref, wu_ref, wi_ref, out_ref):
        j = pl.program_id(0)
        i = pl.program_id(1)
        is_user = i < _NU_BLK
        W = jnp.where(is_user, wu_ref[...], wi_ref[...])
        d = d_ref[...]
        e = ego_ref[...]
        x = jnp.dot(d * e, W, preferred_element_type=jnp.float32) + e
        out_ref[...] = jnp.where(j == 0, x[:, :HD], x[:, HD:])

    return pl.pallas_call(
        body,
        grid=(2, _NBLK),
        in_specs=[
            pl.BlockSpec((_BR, D), lambda j, i: (i, 0)),
            pl.BlockSpec((_BR, 1), lambda j, i: (i, 0)),
            pl.BlockSpec((D, D), lambda j, i: (0, 0)),
            pl.BlockSpec((D, D), lambda j, i: (0, 0)),
        ],
        out_specs=pl.BlockSpec((_BR, HD), lambda j, i: (j * _NBLK + i, 0)),
        out_shape=jax.ShapeDtypeStruct((2 * NN, HD), jnp.float32),
    )(ego, d2, Wu, Wi)


def _stage4(y_cat, ego, gamma, beta, base, nrows):
    g2 = gamma.reshape(1, D)
    b2 = beta.reshape(1, D)
    nblk = nrows // _BR4
    boff = base // _BR4

    def body(ya_ref, yb_ref, ego_ref, g_ref, b_ref, out_ref):
        y = jnp.concatenate([ya_ref[...], yb_ref[...]], axis=1)
        mu = jnp.mean(y, axis=1, keepdims=True)
        var = jnp.mean((y - mu) ** 2, axis=1, keepdims=True)
        out_ref[...] = (g_ref[...] * (y - mu) * lax.rsqrt(var + 1e-5)
                        + b_ref[...] + ego_ref[...])

    return pl.pallas_call(
        body,
        grid=(nblk,),
        in_specs=[
            pl.BlockSpec((_BR4, HD), lambda i: (i + boff, 0)),
            pl.BlockSpec((_BR4, HD), lambda i: (i + boff + _NBLK4, 0)),
            pl.BlockSpec((_BR4, D), lambda i: (i + boff, 0)),
            pl.BlockSpec((1, D), lambda i: (0, 0)),
            pl.BlockSpec((1, D), lambda i: (0, 0)),
        ],
        out_specs=pl.BlockSpec((_BR4, D), lambda i: (i, 0)),
        out_shape=jax.ShapeDtypeStruct((nrows, D), jnp.float32),
    )(y_cat, y_cat, ego, g2, b2)


def kernel(ego_embeddings, adj_indices, adj_values, W_uu, d_uu, par_uu,
           W_ii, d_ii, par_ii, ln_gamma, ln_beta):
    adj = adj_indices.astype(jnp.int32).reshape(2 * NE)
    d_scaled = jnp.concatenate([par_uu[0] * par_uu[1] * d_uu,
                                par_ii[0] * par_ii[1] * d_ii])
    x_cat = _stage1(ego_embeddings, d_scaled, W_uu, W_ii)
    h_cat = _SPMM_F(adj, adj_values, x_cat)
    y_cat = _SPMM_B(adj, adj_values, h_cat)
    out_u = _stage4(y_cat, ego_embeddings, ln_gamma, ln_beta, 0, N_USERS)
    out_i = _stage4(y_cat, ego_embeddings, ln_gamma, ln_beta,
                    N_USERS, N_ITEMS)
    return out_u, out_i


# speedup vs baseline: 1.1175x; 1.1175x over previous
import functools

import jax
import jax.numpy as jnp
from jax import lax
from jax.experimental import pallas as pl
from jax.experimental.pallas import tpu as pltpu
from jax.experimental.pallas import tpu_sc as plsc

N_USERS = 20000
N_ITEMS = 30000
NN = N_USERS + N_ITEMS
D = 64
HD = 32
NE = 800000

NS = 16
CH = 128
NB = NE // CH
KTRIP = 99
ZR = 80
NZB = NN // ZR
ZB_PT = -(-NZB // NS)
WR = 400
NWB = NN // WR
WB_PT = -(-NWB // NS)
NG = CH // 16


def _make_spmm(swap):
    mesh = plsc.VectorSubcoreMesh(core_axis_name="c", subcore_axis_name="s")

    @functools.partial(
        pl.kernel,
        mesh=mesh,
        compiler_params=pltpu.CompilerParams(use_tc_tiling_on_sc=False),
        out_type=jax.ShapeDtypeStruct((2 * NN, HD), jnp.float32),
        scratch_types=[
            pltpu.VMEM_SHARED((NN, HD), jnp.float32),
            pltpu.VMEM((CH,), jnp.int32),
            pltpu.VMEM((CH,), jnp.int32),
            pltpu.VMEM((CH,), jnp.int32),
            pltpu.VMEM((CH,), jnp.int32),
            pltpu.VMEM((CH,), jnp.int32),
            pltpu.VMEM((CH,), jnp.int32),
            pltpu.VMEM((CH,), jnp.int32),
            pltpu.VMEM((CH,), jnp.int32),
            pltpu.VMEM((CH,), jnp.float32),
            pltpu.VMEM((CH,), jnp.float32),
            pltpu.VMEM((CH,), jnp.float32),
            pltpu.VMEM((CH,), jnp.float32),
            pltpu.VMEM((CH, HD), jnp.float32),
            pltpu.VMEM((CH, HD), jnp.float32),
            pltpu.VMEM((CH, HD), jnp.float32),
            pltpu.VMEM((CH, HD), jnp.float32),
            pltpu.VMEM((CH, HD), jnp.float32),
            pltpu.VMEM((CH, HD), jnp.float32),
            pltpu.VMEM((ZR, HD), jnp.float32),
            pltpu.SemaphoreType.DMA,
            pltpu.SemaphoreType.DMA,
            pltpu.SemaphoreType.DMA,
            pltpu.SemaphoreType.DMA,
            pltpu.SemaphoreType.DMA,
            pltpu.SemaphoreType.DMA,
            pltpu.SemaphoreType.DMA,
            pltpu.SemaphoreType.DMA,
            pltpu.SemaphoreType.DMA,
            pltpu.SemaphoreType.DMA,
            pltpu.SemaphoreType.DMA,
            pltpu.SemaphoreType.DMA,
            pltpu.SemaphoreType.DMA,
        ],
    )
    def spmm(adj_hbm, eval_hbm, x_hbm, out_hbm, acc,
             sb0, sb1, sb2, sb3, db0, db1, db2, db3, vb0, vb1, vb2, vb3,
             rw0, rw1, rw2, rw3, ro0, ro1, zb_v,
             si0, si1, si2, si3, sg0, sg1, sg2, sg3, ss0, ss1, ss2, ss3,
             szw):
        c = lax.axis_index("c")
        s = lax.axis_index("s")
        gbase = c * NN
        sbs = [sb0, sb1, sb2, sb3]
        dbs = [db0, db1, db2, db3]
        vbs = [vb0, vb1, vb2, vb3]
        rws = [rw0, rw1, rw2, rw3]
        ros = [ro0, ro1]
        sis = [si0, si1, si2, si3]
        sgs = [sg0, sg1, sg2, sg3]
        sss = [ss0, ss1, ss2, ss3]

        def valid(k):
            return s + NS * k < NB

        srow = 1 if swap else 0
        drow = 0 if swap else 1

        def idx_start(k, q):
            @pl.when(valid(k))
            def _():
                e0 = (s + NS * k) * CH
                pltpu.make_async_copy(adj_hbm.at[pl.ds(srow * NE + e0, CH)],
                                      sbs[q], sis[q]).start()
                pltpu.make_async_copy(adj_hbm.at[pl.ds(drow * NE + e0, CH)],
                                      dbs[q], sis[q]).start()
                pltpu.make_async_copy(eval_hbm.at[pl.ds(e0, CH)],
                                      vbs[q], sis[q]).start()

        def idx_wait_adjust(k, q):
            @pl.when(valid(k))
            def _():
                e0 = (s + NS * k) * CH
                pltpu.make_async_copy(adj_hbm.at[pl.ds(srow * NE + e0, CH)],
                                      sbs[q], sis[q]).wait()
                pltpu.make_async_copy(adj_hbm.at[pl.ds(drow * NE + e0, CH)],
                                      dbs[q], sis[q]).wait()
                pltpu.make_async_copy(eval_hbm.at[pl.ds(e0, CH)],
                                      vbs[q], sis[q]).wait()
                for g in range(NG):
                    sl = pl.ds(g * 16, 16)
                    sbs[q][sl] = sbs[q][sl] + gbase

        def gather_start(k, q):
            @pl.when(valid(k))
            def _():
                pltpu.make_async_copy(x_hbm.at[sbs[q]],
                                      rws[q], sgs[q]).start()

        def gather_wait(k, q):
            @pl.when(valid(k))
            def _():
                pltpu.make_async_copy(x_hbm.at[sbs[q]],
                                      rws[q], sgs[q]).wait()

        def scale(k, q):
            @pl.when(valid(k))
            def _():
                rw = rws[q]
                ro = ros[q % 2]
                vb = vbs[q]
                for g in range(NG):
                    v16 = vb[pl.ds(g * 16, 16)]
                    for i in range(16):
                        r = g * 16 + i
                        vi = v16[i]
                        ro[r, pl.ds(0, 16)] = rw[r, pl.ds(0, 16)] * vi
                        ro[r, pl.ds(16, 16)] = rw[r, pl.ds(16, 16)] * vi

        def scatter_start(k, q):
            @pl.when(valid(k))
            def _():
                pltpu.make_async_copy(ros[q % 2], acc.at[dbs[q]],
                                      sss[q]).start(add=True)

        def scatter_wait(k, q, pred):
            @pl.when(pred)
            def _():
                pltpu.make_async_copy(ros[q % 2], acc.at[dbs[q]],
                                      sss[q]).wait()

        idx_start(0, 0)
        idx_start(1, 1)
        idx_start(2, 2)

        def zz(i, carry):
            zb_v[i, pl.ds(0, 16)] = jnp.zeros((16,), jnp.float32)
            zb_v[i, pl.ds(16, 16)] = jnp.zeros((16,), jnp.float32)
            return carry
        lax.fori_loop(0, ZR, zz, 0)

        def zacc(t, carry):
            b = s + t * NS
            @pl.when(b < NZB)
            def _():
                pltpu.make_async_copy(zb_v, acc.at[pl.ds(b * ZR, ZR)],
                                      szw).start()
            return carry
        lax.fori_loop(0, ZB_PT, zacc, 0)

        def zacc_drain(t, carry):
            b = s + t * NS
            @pl.when(b < NZB)
            def _():
                pltpu.make_async_copy(zb_v, acc.at[pl.ds(b * ZR, ZR)],
                                      szw).wait()
            return carry
        lax.fori_loop(0, ZB_PT, zacc_drain, 0)

        idx_wait_adjust(0, 0)
        gather_start(0, 0)
        idx_wait_adjust(1, 1)
        gather_start(1, 1)
        plsc.subcore_barrier()

        def sub(k, q):
            q1 = (q + 1) % 4
            q2 = (q + 2) % 4
            q3 = (q + 3) % 4
            gather_wait(k, q)
            scale(k, q)
            scatter_start(k, q)
            scatter_wait(k - 1, q3, (k > 0) & valid(k - 1))
            idx_start(k + 3, q3)
            idx_wait_adjust(k + 2, q2)
            gather_start(k + 2, q2)

        def quad(p, carry):
            k = 4 * p
            sub(k, 0)
            sub(k + 1, 1)
            sub(k + 2, 2)
            sub(k + 3, 3)
            return carry
        lax.fori_loop(0, KTRIP, quad, 0)
        plsc.subcore_barrier()

        def wout(t, carry):
            b = s + t * NS
            @pl.when(b < NWB)
            def _():
                pltpu.make_async_copy(acc.at[pl.ds(b * WR, WR)],
                                      out_hbm.at[pl.ds(gbase + b * WR, WR)],
                                      szw).start()
            return carry
        lax.fori_loop(0, WB_PT, wout, 0)

        def wout_drain(t, carry):
            b = s + t * NS
            @pl.when(b < NWB)
            def _():
                pltpu.make_async_copy(acc.at[pl.ds(b * WR, WR)],
                                      out_hbm.at[pl.ds(gbase + b * WR, WR)],
                                      szw).wait()
            return carry
        lax.fori_loop(0, WB_PT, wout_drain, 0)

    return spmm


_SPMM_F = _make_spmm(False)
_SPMM_B = _make_spmm(True)

_BR = 1000
_NBLK = NN // _BR
_NU_BLK = N_USERS // _BR
_BR4 = 2000
_NBLK4 = NN // _BR4


def _stage1(ego, d_scaled, Wu, Wi):
    d2 = d_scaled.reshape(NN, 1)

    def body(ego_ref, d_ref, wu_ref, wi_ref, out_ref):
        j = pl.program_id(0)
        i = pl.program_id(1)
        is_user = i < _NU_BLK
        W = jnp.where(is_user, wu_ref[...], wi_ref[...])
        d = d_ref[...]
        e = ego_ref[...]
        x = jnp.dot(d * e, W, preferred_element_type=jnp.float32) + e
        out_ref[...] = jnp.where(j == 0, x[:, :HD], x[:, HD:])

    return pl.pallas_call(
        body,
        grid=(2, _NBLK),
        in_specs=[
            pl.BlockSpec((_BR, D), lambda j, i: (i, 0)),
            pl.BlockSpec((_BR, 1), lambda j, i: (i, 0)),
            pl.BlockSpec((D, D), lambda j, i: (0, 0)),
            pl.BlockSpec((D, D), lambda j, i: (0, 0)),
        ],
        out_specs=pl.BlockSpec((_BR, HD), lambda j, i: (j * _NBLK + i, 0)),
        out_shape=jax.ShapeDtypeStruct((2 * NN, HD), jnp.float32),
    )(ego, d2, Wu, Wi)


def _stage4(y_cat, ego, gamma, beta, base, nrows):
    g2 = gamma.reshape(1, D)
    b2 = beta.reshape(1, D)
    nblk = nrows // _BR4
    boff = base // _BR4

    def body(ya_ref, yb_ref, ego_ref, g_ref, b_ref, out_ref):
        y = jnp.concatenate([ya_ref[...], yb_ref[...]], axis=1)
        mu = jnp.mean(y, axis=1, keepdims=True)
        var = jnp.mean((y - mu) ** 2, axis=1, keepdims=True)
        out_ref[...] = (g_ref[...] * (y - mu) * lax.rsqrt(var + 1e-5)
                        + b_ref[...] + ego_ref[...])

    return pl.pallas_call(
        body,
        grid=(nblk,),
        in_specs=[
            pl.BlockSpec((_BR4, HD), lambda i: (i + boff, 0)),
            pl.BlockSpec((_BR4, HD), lambda i: (i + boff + _NBLK4, 0)),
            pl.BlockSpec((_BR4, D), lambda i: (i + boff, 0)),
            pl.BlockSpec((1, D), lambda i: (0, 0)),
            pl.BlockSpec((1, D), lambda i: (0, 0)),
        ],
        out_specs=pl.BlockSpec((_BR4, D), lambda i: (i, 0)),
        out_shape=jax.ShapeDtypeStruct((nrows, D), jnp.float32),
    )(y_cat, y_cat, ego, g2, b2)


def kernel(ego_embeddings, adj_indices, adj_values, W_uu, d_uu, par_uu,
           W_ii, d_ii, par_ii, ln_gamma, ln_beta):
    adj = adj_indices.astype(jnp.int32).reshape(2 * NE)
    d_scaled = jnp.concatenate([par_uu[0] * par_uu[1] * d_uu,
                                par_ii[0] * par_ii[1] * d_ii])
    x_cat = _stage1(ego_embeddings, d_scaled, W_uu, W_ii)
    h_cat = _SPMM_F(adj, adj_values, x_cat)
    y_cat = _SPMM_B(adj, adj_values, h_cat)
    out_u = _stage4(y_cat, ego_embeddings, ln_gamma, ln_beta, 0, N_USERS)
    out_i = _stage4(y_cat, ego_embeddings, ln_gamma, ln_beta,
                    N_USERS, N_ITEMS)
    return out_u, out_i


# speedup vs baseline: 1.1816x; 1.0574x over previous
import functools

import jax
import jax.numpy as jnp
from jax import lax
from jax.experimental import pallas as pl
from jax.experimental.pallas import tpu as pltpu
from jax.experimental.pallas import tpu_sc as plsc

N_USERS = 20000
N_ITEMS = 30000
NN = N_USERS + N_ITEMS
D = 64
HD = 32
NE = 800000

NS = 16
CH = 128
NB = NE // CH
KTRIP = 99
ZR = 80
NZB = NN // ZR
ZB_PT = -(-NZB // NS)
WR = 400
NWB = NN // WR
WB_PT = -(-NWB // NS)
NG = CH // 16


def _make_spmm(swap):
    mesh = plsc.VectorSubcoreMesh(core_axis_name="c", subcore_axis_name="s")

    @functools.partial(
        pl.kernel,
        mesh=mesh,
        compiler_params=pltpu.CompilerParams(use_tc_tiling_on_sc=False),
        out_type=(jax.ShapeDtypeStruct((NN, HD), jnp.float32),
                  jax.ShapeDtypeStruct((NN, HD), jnp.float32)),
        scratch_types=[
            pltpu.VMEM_SHARED((NN, HD), jnp.float32),
            pltpu.VMEM((CH,), jnp.int32),
            pltpu.VMEM((CH,), jnp.int32),
            pltpu.VMEM((CH,), jnp.int32),
            pltpu.VMEM((CH,), jnp.int32),
            pltpu.VMEM((CH,), jnp.int32),
            pltpu.VMEM((CH,), jnp.int32),
            pltpu.VMEM((CH,), jnp.int32),
            pltpu.VMEM((CH,), jnp.int32),
            pltpu.VMEM((CH,), jnp.float32),
            pltpu.VMEM((CH,), jnp.float32),
            pltpu.VMEM((CH,), jnp.float32),
            pltpu.VMEM((CH,), jnp.float32),
            pltpu.VMEM((CH, HD), jnp.float32),
            pltpu.VMEM((CH, HD), jnp.float32),
            pltpu.VMEM((CH, HD), jnp.float32),
            pltpu.VMEM((CH, HD), jnp.float32),
            pltpu.VMEM((CH, HD), jnp.float32),
            pltpu.VMEM((CH, HD), jnp.float32),
            pltpu.VMEM((ZR, HD), jnp.float32),
            pltpu.SemaphoreType.DMA,
            pltpu.SemaphoreType.DMA,
            pltpu.SemaphoreType.DMA,
            pltpu.SemaphoreType.DMA,
            pltpu.SemaphoreType.DMA,
            pltpu.SemaphoreType.DMA,
            pltpu.SemaphoreType.DMA,
            pltpu.SemaphoreType.DMA,
            pltpu.SemaphoreType.DMA,
            pltpu.SemaphoreType.DMA,
            pltpu.SemaphoreType.DMA,
            pltpu.SemaphoreType.DMA,
            pltpu.SemaphoreType.DMA,
        ],
    )
    def spmm(adj_hbm, eval_hbm, xlo_hbm, xhi_hbm, outlo_hbm, outhi_hbm, acc,
             sb0, sb1, sb2, sb3, db0, db1, db2, db3, vb0, vb1, vb2, vb3,
             rw0, rw1, rw2, rw3, ro0, ro1, zb_v,
             si0, si1, si2, si3, sg0, sg1, sg2, sg3, ss0, ss1, ss2, ss3,
             szw):
        c = lax.axis_index("c")
        s = lax.axis_index("s")
        sbs = [sb0, sb1, sb2, sb3]
        dbs = [db0, db1, db2, db3]
        vbs = [vb0, vb1, vb2, vb3]
        rws = [rw0, rw1, rw2, rw3]
        ros = [ro0, ro1]
        sis = [si0, si1, si2, si3]
        sgs = [sg0, sg1, sg2, sg3]
        sss = [ss0, ss1, ss2, ss3]

        def valid(k):
            return s + NS * k < NB

        srow = 1 if swap else 0
        drow = 0 if swap else 1

        def idx_start(k, q):
            @pl.when(valid(k))
            def _():
                e0 = (s + NS * k) * CH
                pltpu.make_async_copy(adj_hbm.at[pl.ds(srow * NE + e0, CH)],
                                      sbs[q], sis[q]).start()
                pltpu.make_async_copy(adj_hbm.at[pl.ds(drow * NE + e0, CH)],
                                      dbs[q], sis[q]).start()
                pltpu.make_async_copy(eval_hbm.at[pl.ds(e0, CH)],
                                      vbs[q], sis[q]).start()

        def idx_wait_adjust(k, q):
            @pl.when(valid(k))
            def _():
                e0 = (s + NS * k) * CH
                pltpu.make_async_copy(adj_hbm.at[pl.ds(srow * NE + e0, CH)],
                                      sbs[q], sis[q]).wait()
                pltpu.make_async_copy(adj_hbm.at[pl.ds(drow * NE + e0, CH)],
                                      dbs[q], sis[q]).wait()
                pltpu.make_async_copy(eval_hbm.at[pl.ds(e0, CH)],
                                      vbs[q], sis[q]).wait()

        def gather_start(k, q):
            @pl.when(valid(k) & (c == 0))
            def _():
                pltpu.make_async_copy(xlo_hbm.at[sbs[q]],
                                      rws[q], sgs[q]).start()
            @pl.when(valid(k) & (c == 1))
            def _():
                pltpu.make_async_copy(xhi_hbm.at[sbs[q]],
                                      rws[q], sgs[q]).start()

        def gather_wait(k, q):
            @pl.when(valid(k) & (c == 0))
            def _():
                pltpu.make_async_copy(xlo_hbm.at[sbs[q]],
                                      rws[q], sgs[q]).wait()
            @pl.when(valid(k) & (c == 1))
            def _():
                pltpu.make_async_copy(xhi_hbm.at[sbs[q]],
                                      rws[q], sgs[q]).wait()

        def scale(k, q):
            @pl.when(valid(k))
            def _():
                rw = rws[q]
                ro = ros[q % 2]
                vb = vbs[q]
                for g in range(NG):
                    v16 = vb[pl.ds(g * 16, 16)]
                    for i in range(16):
                        r = g * 16 + i
                        vi = v16[i]
                        ro[r, pl.ds(0, 16)] = rw[r, pl.ds(0, 16)] * vi
                        ro[r, pl.ds(16, 16)] = rw[r, pl.ds(16, 16)] * vi

        def scatter_start(k, q):
            @pl.when(valid(k))
            def _():
                pltpu.make_async_copy(ros[q % 2], acc.at[dbs[q]],
                                      sss[q]).start(add=True)

        def scatter_wait(k, q, pred):
            @pl.when(pred)
            def _():
                pltpu.make_async_copy(ros[q % 2], acc.at[dbs[q]],
                                      sss[q]).wait()

        idx_start(0, 0)
        idx_start(1, 1)
        idx_start(2, 2)

        def zz(i, carry):
            zb_v[i, pl.ds(0, 16)] = jnp.zeros((16,), jnp.float32)
            zb_v[i, pl.ds(16, 16)] = jnp.zeros((16,), jnp.float32)
            return carry
        lax.fori_loop(0, ZR, zz, 0)

        def zacc(t, carry):
            b = s + t * NS
            @pl.when(b < NZB)
            def _():
                pltpu.make_async_copy(zb_v, acc.at[pl.ds(b * ZR, ZR)],
                                      szw).start()
            return carry
        lax.fori_loop(0, ZB_PT, zacc, 0)

        def zacc_drain(t, carry):
            b = s + t * NS
            @pl.when(b < NZB)
            def _():
                pltpu.make_async_copy(zb_v, acc.at[pl.ds(b * ZR, ZR)],
                                      szw).wait()
            return carry
        lax.fori_loop(0, ZB_PT, zacc_drain, 0)

        idx_wait_adjust(0, 0)
        gather_start(0, 0)
        idx_wait_adjust(1, 1)
        gather_start(1, 1)
        plsc.subcore_barrier()

        def sub(k, q):
            q1 = (q + 1) % 4
            q2 = (q + 2) % 4
            q3 = (q + 3) % 4
            gather_wait(k, q)
            scale(k, q)
            scatter_start(k, q)
            scatter_wait(k - 1, q3, (k > 0) & valid(k - 1))
            idx_start(k + 3, q3)
            idx_wait_adjust(k + 2, q2)
            gather_start(k + 2, q2)

        def quad(p, carry):
            k = 4 * p
            sub(k, 0)
            sub(k + 1, 1)
            sub(k + 2, 2)
            sub(k + 3, 3)
            return carry
        lax.fori_loop(0, KTRIP, quad, 0)
        plsc.subcore_barrier()

        def wout(t, carry):
            b = s + t * NS
            @pl.when((b < NWB) & (c == 0))
            def _():
                pltpu.make_async_copy(acc.at[pl.ds(b * WR, WR)],
                                      outlo_hbm.at[pl.ds(b * WR, WR)],
                                      szw).start()
            @pl.when((b < NWB) & (c == 1))
            def _():
                pltpu.make_async_copy(acc.at[pl.ds(b * WR, WR)],
                                      outhi_hbm.at[pl.ds(b * WR, WR)],
                                      szw).start()
            return carry
        lax.fori_loop(0, WB_PT, wout, 0)

        def wout_drain(t, carry):
            b = s + t * NS
            @pl.when((b < NWB) & (c == 0))
            def _():
                pltpu.make_async_copy(acc.at[pl.ds(b * WR, WR)],
                                      outlo_hbm.at[pl.ds(b * WR, WR)],
                                      szw).wait()
            @pl.when((b < NWB) & (c == 1))
            def _():
                pltpu.make_async_copy(acc.at[pl.ds(b * WR, WR)],
                                      outhi_hbm.at[pl.ds(b * WR, WR)],
                                      szw).wait()
            return carry
        lax.fori_loop(0, WB_PT, wout_drain, 0)

    return spmm


_SPMM_F = _make_spmm(False)
_SPMM_B = _make_spmm(True)

_BR = 1000
_NBLK = NN // _BR
_NU_BLK = N_USERS // _BR
_BR4 = 2000
_NBLK4 = NN // _BR4


def _stage1(ego, d_scaled, Wu, Wi):
    d2 = d_scaled.reshape(NN, 1)

    def body(ego_ref, d_ref, wu_ref, wi_ref, lo_ref, hi_ref):
        i = pl.program_id(0)
        is_user = i < _NU_BLK
        W = jnp.where(is_user, wu_ref[...], wi_ref[...])
        d = d_ref[...]
        e = ego_ref[...]
        x = jnp.dot(d * e, W, preferred_element_type=jnp.float32) + e
        lo_ref[...] = x[:, :HD]
        hi_ref[...] = x[:, HD:]

    return pl.pallas_call(
        body,
        grid=(_NBLK,),
        in_specs=[
            pl.BlockSpec((_BR, D), lambda i: (i, 0)),
            pl.BlockSpec((_BR, 1), lambda i: (i, 0)),
            pl.BlockSpec((D, D), lambda i: (0, 0)),
            pl.BlockSpec((D, D), lambda i: (0, 0)),
        ],
        out_specs=[pl.BlockSpec((_BR, HD), lambda i: (i, 0)),
                   pl.BlockSpec((_BR, HD), lambda i: (i, 0))],
        out_shape=[jax.ShapeDtypeStruct((NN, HD), jnp.float32),
                   jax.ShapeDtypeStruct((NN, HD), jnp.float32)],
    )(ego, d2, Wu, Wi)


def _stage4(y_lo, y_hi, ego, gamma, beta, base, nrows):
    g2 = gamma.reshape(1, D)
    b2 = beta.reshape(1, D)
    nblk = nrows // _BR4
    boff = base // _BR4

    def body(ya_ref, yb_ref, ego_ref, g_ref, b_ref, out_ref):
        y = jnp.concatenate([ya_ref[...], yb_ref[...]], axis=1)
        mu = jnp.mean(y, axis=1, keepdims=True)
        var = jnp.mean((y - mu) ** 2, axis=1, keepdims=True)
        out_ref[...] = (g_ref[...] * (y - mu) * lax.rsqrt(var + 1e-5)
                        + b_ref[...] + ego_ref[...])

    return pl.pallas_call(
        body,
        grid=(nblk,),
        in_specs=[
            pl.BlockSpec((_BR4, HD), lambda i: (i + boff, 0)),
            pl.BlockSpec((_BR4, HD), lambda i: (i + boff, 0)),
            pl.BlockSpec((_BR4, D), lambda i: (i + boff, 0)),
            pl.BlockSpec((1, D), lambda i: (0, 0)),
            pl.BlockSpec((1, D), lambda i: (0, 0)),
        ],
        out_specs=pl.BlockSpec((_BR4, D), lambda i: (i, 0)),
        out_shape=jax.ShapeDtypeStruct((nrows, D), jnp.float32),
    )(y_lo, y_hi, ego, g2, b2)


def kernel(ego_embeddings, adj_indices, adj_values, W_uu, d_uu, par_uu,
           W_ii, d_ii, par_ii, ln_gamma, ln_beta):
    adj = adj_indices.astype(jnp.int32).reshape(2 * NE)
    d_scaled = jnp.concatenate([par_uu[0] * par_uu[1] * d_uu,
                                par_ii[0] * par_ii[1] * d_ii])
    x_lo, x_hi = _stage1(ego_embeddings, d_scaled, W_uu, W_ii)
    h_lo, h_hi = _SPMM_F(adj, adj_values, x_lo, x_hi)
    y_lo, y_hi = _SPMM_B(adj, adj_values, h_lo, h_hi)
    out_u = _stage4(y_lo, y_hi, ego_embeddings, ln_gamma, ln_beta,
                    0, N_USERS)
    out_i = _stage4(y_lo, y_hi, ego_embeddings, ln_gamma, ln_beta,
                    N_USERS, N_ITEMS)
    return out_u, out_i
